# baseline (speedup 1.0000x reference)
import functools

import jax
import jax.numpy as jnp
from jax import lax
from jax.experimental import pallas as pl
from jax.experimental.pallas import tpu as pltpu
from jax.experimental.pallas import tpu_sc as plsc


def kernel(x, table):
    B, S, D = x.shape
    N = B * S * D
    xf = x.reshape(N)
    tf = table.reshape(S * D)
    NW = 32
    rows_per_w = (B * S) // NW
    C = 16
    chunks = rows_per_w // C
    CHUNK = C * D

    mesh = plsc.VectorSubcoreMesh(core_axis_name="c", subcore_axis_name="s")

    @functools.partial(
        pl.kernel, mesh=mesh,
        out_type=jax.ShapeDtypeStruct((N,), jnp.float32),
        scratch_types=[
            pltpu.VMEM((CHUNK,), jnp.float32),
            pltpu.VMEM((CHUNK,), jnp.float32),
        ],
    )
    def k(xf_hbm, tf_hbm, out_hbm, xb, tb):
        w = lax.axis_index("s") * 2 + lax.axis_index("c")
        xbase = w * (rows_per_w * D)
        tbase = xbase % (S * D)

        def chunk_body(g, _):
            off = g * CHUNK
            pltpu.sync_copy(xf_hbm.at[pl.ds(xbase + off, CHUNK)], xb)
            pltpu.sync_copy(tf_hbm.at[pl.ds(tbase + off, CHUNK)], tb)

            def add_body(i, _):
                sl = pl.ds(i * 16, 16)
                xb[sl] = xb[sl] + tb[sl]
                return 0

            lax.fori_loop(0, CHUNK // 16, add_body, 0, unroll=8)
            pltpu.sync_copy(xb, out_hbm.at[pl.ds(xbase + off, CHUNK)])
            return 0

        lax.fori_loop(0, chunks, chunk_body, 0)

    return k(xf, tf).reshape(B, S, D)


# speedup vs baseline: 2.0857x; 2.0857x over previous
import functools

import jax
import jax.numpy as jnp
from jax import lax
from jax.experimental import pallas as pl
from jax.experimental.pallas import tpu as pltpu
from jax.experimental.pallas import tpu_sc as plsc


def kernel(x, table):
    B, S, D = x.shape
    NW = 32
    rows_per_w = (B * S) // NW
    wpb = S // rows_per_w
    C = 16
    chunks = rows_per_w // C
    NBUF = 2

    mesh = plsc.VectorSubcoreMesh(core_axis_name="c", subcore_axis_name="s")

    @functools.partial(
        pl.kernel, mesh=mesh,
        out_type=jax.ShapeDtypeStruct((B, S, D), jnp.float32),
        scratch_types=[
            pltpu.VMEM((NBUF, C, D), jnp.float32),
            pltpu.VMEM((NBUF, C, D), jnp.float32),
            pltpu.VMEM((NBUF, C, D), jnp.float32),
            pltpu.SemaphoreType.DMA((NBUF,)),
            pltpu.SemaphoreType.DMA((NBUF,)),
            pltpu.SemaphoreType.DMA((NBUF,)),
        ],
    )
    def k(x_hbm, t_hbm, out_hbm, xb, tb, ob, sx, st, so):
        w = lax.axis_index("s") * 2 + lax.axis_index("c")
        bw = w // wpb
        row0 = (w % wpb) * rows_per_w

        def in_copies(slot, g):
            r0 = row0 + g * C
            cx = pltpu.make_async_copy(
                x_hbm.at[bw, pl.ds(r0, C), :], xb.at[slot], sx.at[slot])
            ct = pltpu.make_async_copy(
                t_hbm.at[pl.ds(r0, C), :], tb.at[slot], st.at[slot])
            return cx, ct

        def out_copy(slot, g):
            r0 = row0 + g * C
            return pltpu.make_async_copy(
                ob.at[slot], out_hbm.at[bw, pl.ds(r0, C), :], so.at[slot])

        for b in range(NBUF):
            cx, ct = in_copies(b, b)
            cx.start()
            ct.start()

        def outer(i, _):
            g0 = i * NBUF
            for b in range(NBUF):
                g = g0 + b
                cx, ct = in_copies(b, g)
                cx.wait()
                ct.wait()

                @pl.when(g >= NBUF)
                def _():
                    out_copy(b, g - NBUF).wait()

                def row_body(r, _):
                    def col_body(c, _):
                        sl = pl.ds(c * 16, 16)
                        ob[b, r, sl] = xb[b, r, sl] + tb[b, r, sl]
                        return 0
                    lax.fori_loop(0, D // 16, col_body, 0, unroll=8)
                    return 0

                lax.fori_loop(0, C, row_body, 0)
                out_copy(b, g).start()

                @pl.when(g + NBUF < chunks)
                def _():
                    cx2, ct2 = in_copies(b, g + NBUF)
                    cx2.start()
                    ct2.start()
            return 0

        lax.fori_loop(0, chunks // NBUF, outer, 0)
        for b in range(NBUF):
            out_copy(b, chunks - NBUF + b).wait()

    return k(x, table)


# speedup vs baseline: 4.9882x; 2.3916x over previous
import functools

import jax
import jax.numpy as jnp
from jax import lax
from jax.experimental import pallas as pl
from jax.experimental.pallas import tpu as pltpu
from jax.experimental.pallas import tpu_sc as plsc


def kernel(x, table):
    B, S, D = x.shape
    NW = 32
    rows_per_w = (B * S) // NW
    wpb = S // rows_per_w
    C = 16
    chunks = rows_per_w // C
    NBUF = 2

    mesh = plsc.VectorSubcoreMesh(core_axis_name="c", subcore_axis_name="s")

    @functools.partial(
        pl.kernel, mesh=mesh,
        out_type=jax.ShapeDtypeStruct((B, S, D), jnp.float32),
        scratch_types=[
            pltpu.VMEM((NBUF, C, D), jnp.float32),
            pltpu.VMEM((NBUF, C, D), jnp.float32),
            pltpu.VMEM((NBUF, C, D), jnp.float32),
            pltpu.SemaphoreType.DMA((NBUF,)),
            pltpu.SemaphoreType.DMA((NBUF,)),
            pltpu.SemaphoreType.DMA((NBUF,)),
        ],
    )
    def k(x_hbm, t_hbm, out_hbm, xb, tb, ob, sx, st, so):
        w = lax.axis_index("s") * 2 + lax.axis_index("c")
        bw = w // wpb
        row0 = (w % wpb) * rows_per_w

        def in_copies(slot, g):
            r0 = row0 + g * C
            cx = pltpu.make_async_copy(
                x_hbm.at[bw, pl.ds(r0, C), :], xb.at[slot], sx.at[slot])
            ct = pltpu.make_async_copy(
                t_hbm.at[pl.ds(r0, C), :], tb.at[slot], st.at[slot])
            return cx, ct

        def out_copy(slot, g):
            r0 = row0 + g * C
            return pltpu.make_async_copy(
                ob.at[slot], out_hbm.at[bw, pl.ds(r0, C), :], so.at[slot])

        for b in range(NBUF):
            cx, ct = in_copies(b, b)
            cx.start()
            ct.start()

        def outer(i, _):
            g0 = i * NBUF
            for b in range(NBUF):
                g = g0 + b
                cx, ct = in_copies(b, g)
                cx.wait()
                ct.wait()

                @pl.when(g >= NBUF)
                def _():
                    out_copy(b, g - NBUF).wait()

                vecs_per_row = D // 16

                @plsc.parallel_loop(0, C * vecs_per_row, unroll=8)
                def add_loop(i):
                    r = i // vecs_per_row
                    sl = pl.ds((i % vecs_per_row) * 16, 16)
                    ob[b, r, sl] = xb[b, r, sl] + tb[b, r, sl]
                out_copy(b, g).start()

                @pl.when(g + NBUF < chunks)
                def _():
                    cx2, ct2 = in_copies(b, g + NBUF)
                    cx2.start()
                    ct2.start()
            return 0

        lax.fori_loop(0, chunks // NBUF, outer, 0)
        for b in range(NBUF):
            out_copy(b, chunks - NBUF + b).wait()

    return k(x, table)


# speedup vs baseline: 5.1014x; 1.0227x over previous
import functools

import jax
import jax.numpy as jnp
from jax import lax
from jax.experimental import pallas as pl
from jax.experimental.pallas import tpu as pltpu
from jax.experimental.pallas import tpu_sc as plsc


def kernel(x, table):
    B, S, D = x.shape
    NW = 32
    rows_per_w = (B * S) // NW
    wpb = S // rows_per_w
    C = 8
    chunks = rows_per_w // C
    NBUF = 4

    mesh = plsc.VectorSubcoreMesh(core_axis_name="c", subcore_axis_name="s")

    @functools.partial(
        pl.kernel, mesh=mesh,
        out_type=jax.ShapeDtypeStruct((B, S, D), jnp.float32),
        scratch_types=[
            pltpu.VMEM((NBUF, C, D), jnp.float32),
            pltpu.VMEM((NBUF, C, D), jnp.float32),
            pltpu.VMEM((NBUF, C, D), jnp.float32),
            pltpu.SemaphoreType.DMA((NBUF,)),
            pltpu.SemaphoreType.DMA((NBUF,)),
            pltpu.SemaphoreType.DMA((NBUF,)),
        ],
    )
    def k(x_hbm, t_hbm, out_hbm, xb, tb, ob, sx, st, so):
        w = lax.axis_index("s") * 2 + lax.axis_index("c")
        bw = w // wpb
        row0 = (w % wpb) * rows_per_w

        def in_copies(slot, g):
            r0 = row0 + g * C
            cx = pltpu.make_async_copy(
                x_hbm.at[bw, pl.ds(r0, C), :], xb.at[slot], sx.at[slot])
            ct = pltpu.make_async_copy(
                t_hbm.at[pl.ds(r0, C), :], tb.at[slot], st.at[slot])
            return cx, ct

        def out_copy(slot, g):
            r0 = row0 + g * C
            return pltpu.make_async_copy(
                ob.at[slot], out_hbm.at[bw, pl.ds(r0, C), :], so.at[slot])

        for b in range(NBUF):
            cx, ct = in_copies(b, b)
            cx.start()
            ct.start()

        def outer(i, _):
            g0 = i * NBUF
            for b in range(NBUF):
                g = g0 + b
                cx, ct = in_copies(b, g)
                cx.wait()
                ct.wait()

                @pl.when(g >= NBUF)
                def _():
                    out_copy(b, g - NBUF).wait()

                vecs_per_row = D // 16

                @plsc.parallel_loop(0, C * vecs_per_row, unroll=8)
                def add_loop(i):
                    r = i // vecs_per_row
                    sl = pl.ds((i % vecs_per_row) * 16, 16)
                    ob[b, r, sl] = xb[b, r, sl] + tb[b, r, sl]
                out_copy(b, g).start()

                @pl.when(g + NBUF < chunks)
                def _():
                    cx2, ct2 = in_copies(b, g + NBUF)
                    cx2.start()
                    ct2.start()
            return 0

        lax.fori_loop(0, chunks // NBUF, outer, 0)
        for b in range(NBUF):
            out_copy(b, chunks - NBUF + b).wait()

    return k(x, table)
